# Initial kernel scaffold; baseline (speedup 1.0000x reference)
import jax
import jax.numpy as jnp
from jax.experimental import pallas as pl


def kernel(x):
    raise NotImplementedError("write your pallas kernel here")


# speedup vs baseline: 7.7033x; 7.7033x over previous
import functools

import jax
import jax.numpy as jnp
from jax import lax
from jax.experimental import pallas as pl
from jax.experimental.pallas import tpu as pltpu
from jax.experimental.pallas import tpu_sc as plsc

N = 8388608
NHALF = N // 2
PHASE_MASK = N - 1
NFREQ = NHALF
NW = 32
PER_W = NFREQ // NW
DMA_CHUNK = 16384
N_DMA = PER_W // DMA_CHUNK
SUB = 512
N_SUB = DMA_CHUNK // SUB
VEC = 16
N_VEC = SUB // VEC
TWO_PI_OVER_N = 2.0 * 3.14159265358979323846 / N


def _merge_pool(vm, vi, vre, vim, poolm, pooli, poolre, poolim, lane):
  nk, si, sre, sim = lax.sort((-vm, vi, vre, vim), num_keys=1)
  sm = -nk
  sel = lane < 8
  cm = jnp.where(sel, poolm, lax.rev(sm, (0,)))
  ci = jnp.where(sel, pooli, lax.rev(si, (0,)))
  cre = jnp.where(sel, poolre, lax.rev(sre, (0,)))
  cim = jnp.where(sel, poolim, lax.rev(sim, (0,)))
  nk2, pi2, pre2, pim2 = lax.sort((-cm, ci, cre, cim), num_keys=1)
  pm2 = -nk2
  new_t = jnp.min(jnp.where(lane < 5, pm2, jnp.float32(3e38)))
  return pm2, pi2, pre2, pim2, new_t


def _sc_topk_body(re_hbm, im_hbm, outm, outi, outre, outim,
                  re_buf, im_buf, stg_m, stg_i, stg_re, stg_im):
  wid = lax.axis_index("s") * 2 + lax.axis_index("c")
  base = wid * PER_W
  lane = lax.iota(jnp.int32, 16)

  state0 = (
      jnp.full((VEC,), -1.0, jnp.float32),
      jnp.zeros((VEC,), jnp.int32),
      jnp.zeros((VEC,), jnp.float32),
      jnp.zeros((VEC,), jnp.float32),
      jnp.float32(-1.0),
  )

  def process_sub(c_off, g_off, state):
    def maxbody(v, acc):
      off = c_off + v * VEC
      rr = re_buf[pl.ds(off, VEC)]
      ii = im_buf[pl.ds(off, VEC)]
      return jnp.maximum(acc, rr * rr + ii * ii)

    m_acc = lax.fori_loop(0, N_VEC, maxbody, jnp.full((VEC,), -2.0, jnp.float32))
    sub_max = jnp.max(m_acc)

    def rescan(st):
      def body(v, st2):
        off = c_off + v * VEC
        rr = re_buf[pl.ds(off, VEC)]
        ii = im_buf[pl.ds(off, VEC)]
        vm = rr * rr + ii * ii
        vmax = jnp.max(vm)

        def do_merge(st3):
          pm, pi, pre, pim, _t = st3
          vi = lane + (g_off + off + 1)
          return _merge_pool(vm, vi, rr, ii, pm, pi, pre, pim, lane)

        return lax.cond(vmax > st2[4], do_merge, lambda s: s, st2)

      return lax.fori_loop(0, N_VEC, body, st)

    return lax.cond(sub_max > state[4], rescan, lambda s: s, state)

  def dma_step(s, state):
    start = base + s * DMA_CHUNK
    pltpu.sync_copy(re_hbm.at[pl.ds(start, DMA_CHUNK)], re_buf)
    pltpu.sync_copy(im_hbm.at[pl.ds(start, DMA_CHUNK)], im_buf)

    def sub(c, st):
      return process_sub(c * SUB, s * DMA_CHUNK + base, st)

    return lax.fori_loop(0, N_SUB, sub, state)

  poolm, pooli, poolre, poolim, _t = lax.fori_loop(0, N_DMA, dma_step, state0)

  stg_m[...] = poolm
  stg_i[...] = pooli
  stg_re[...] = poolre
  stg_im[...] = poolim
  pltpu.sync_copy(stg_m, outm.at[pl.ds(wid * VEC, VEC)])
  pltpu.sync_copy(stg_i, outi.at[pl.ds(wid * VEC, VEC)])
  pltpu.sync_copy(stg_re, outre.at[pl.ds(wid * VEC, VEC)])
  pltpu.sync_copy(stg_im, outim.at[pl.ds(wid * VEC, VEC)])


@functools.cache
def _sc_topk():
  return pl.kernel(
      _sc_topk_body,
      out_type=[
          jax.ShapeDtypeStruct((NW * VEC,), jnp.float32),
          jax.ShapeDtypeStruct((NW * VEC,), jnp.int32),
          jax.ShapeDtypeStruct((NW * VEC,), jnp.float32),
          jax.ShapeDtypeStruct((NW * VEC,), jnp.float32),
      ],
      mesh=plsc.VectorSubcoreMesh(core_axis_name="c", subcore_axis_name="s"),
      compiler_params=pltpu.CompilerParams(needs_layout_passes=False),
      scratch_types=[
          pltpu.VMEM((DMA_CHUNK,), jnp.float32),
          pltpu.VMEM((DMA_CHUNK,), jnp.float32),
          pltpu.VMEM((VEC,), jnp.float32),
          pltpu.VMEM((VEC,), jnp.int32),
          pltpu.VMEM((VEC,), jnp.float32),
          pltpu.VMEM((VEC,), jnp.float32),
      ],
  )


ROWS = 8192
COLS = 1024
BLK_R = 256
GRID = ROWS // BLK_R


def _tc_synth_body(candm_ref, candi_ref, candre_ref, candim_ref, x_ref,
                   season_ref, trend_ref):
  m = candm_ref[...]
  ci = candi_ref[...]
  cre = candre_ref[...]
  cim = candim_ref[...]
  fi = (lax.broadcasted_iota(jnp.int32, m.shape, 0) * m.shape[1]
        + lax.broadcasted_iota(jnp.int32, m.shape, 1))
  avail = fi >= 0
  vals = []
  for _ in range(5):
    cur = jnp.where(avail, m, jnp.float32(-3.0))
    mk = jnp.max(cur)
    pick = jnp.min(jnp.where(cur == mk, fi, jnp.int32(1 << 30)))
    sel = fi == pick
    jk = jnp.sum(jnp.where(sel, ci, 0))
    rek = jnp.sum(jnp.where(sel, cre, jnp.float32(0.0)))
    imk = jnp.sum(jnp.where(sel, cim, jnp.float32(0.0)))
    avail = jnp.logical_and(avail, jnp.logical_not(sel))
    vals.append((mk, jk, rek, imk))
  thresh2 = vals[4][0]

  i = pl.program_id(0)
  n1 = i * BLK_R + lax.broadcasted_iota(jnp.int32, (BLK_R, 1), 0)
  n2 = lax.broadcasted_iota(jnp.int32, (1, COLS), 1)
  season = jnp.zeros((BLK_R, COLS), jnp.float32)
  for k in range(5):
    mk, jk, rek, imk = vals[k]
    alive = mk > thresh2
    is_nyq = jk == NHALF
    w = jnp.where(is_nyq, jnp.float32(1.0), jnp.float32(2.0)) * jnp.float32(1.0 / N)
    a = jnp.where(alive, w * rek, jnp.float32(0.0))
    b = jnp.where(jnp.logical_and(alive, jnp.logical_not(is_nyq)),
                  -w * imk, jnp.float32(0.0))
    m1 = (jk * (n1 * COLS)) & PHASE_MASK
    m2 = (jk * n2) & PHASE_MASK
    th1 = m1.astype(jnp.float32) * jnp.float32(TWO_PI_OVER_N)
    th2 = m2.astype(jnp.float32) * jnp.float32(TWO_PI_OVER_N)
    c1 = jnp.cos(th1)
    s1 = jnp.sin(th1)
    c2 = jnp.cos(th2)
    s2 = jnp.sin(th2)
    p = a * c1 + b * s1
    q = b * c1 - a * s1
    season = season + p * c2 + q * s2
  xv = x_ref[...]
  season_ref[...] = season
  trend_ref[...] = xv - season


def _tc_synth(candm, candi, candre, candim, x2):
  cand_spec = pl.BlockSpec((4, 128), lambda i: (0, 0))
  return pl.pallas_call(
      _tc_synth_body,
      grid=(GRID,),
      in_specs=[cand_spec, cand_spec, cand_spec, cand_spec,
                pl.BlockSpec((BLK_R, COLS), lambda i: (i, 0))],
      out_specs=[pl.BlockSpec((BLK_R, COLS), lambda i: (i, 0)),
                 pl.BlockSpec((BLK_R, COLS), lambda i: (i, 0))],
      out_shape=[jax.ShapeDtypeStruct((ROWS, COLS), jnp.float32),
                 jax.ShapeDtypeStruct((ROWS, COLS), jnp.float32)],
  )(candm, candi, candre, candim, x2)


def kernel(x):
  xf = jnp.fft.rfft(x)
  re1 = jnp.real(xf)[1:]
  im1 = jnp.imag(xf)[1:]
  candm, candi, candre, candim = _sc_topk()(re1, im1)
  x2 = x.reshape(ROWS, COLS)
  season, trend = _tc_synth(candm.reshape(4, 128), candi.reshape(4, 128),
                            candre.reshape(4, 128), candim.reshape(4, 128), x2)
  return season.reshape(-1), trend.reshape(-1)


# speedup vs baseline: 11.0057x; 1.4287x over previous
import functools

import numpy as np

import jax
import jax.numpy as jnp
from jax import lax
from jax.experimental import pallas as pl
from jax.experimental.pallas import tpu as pltpu
from jax.experimental.pallas import tpu_sc as plsc

N = 8388608
NHALF = N // 2
PHASE_MASK = N - 1
TWO_PI_OVER_N = 2.0 * np.pi / N

R1 = 256
R2 = 256
R3 = 128
N23 = R2 * R3

PREC = jax.lax.Precision.HIGHEST


def _dft_tables(r):
  k = np.arange(r)
  ph = 2.0 * np.pi * (np.outer(k, k) % r).astype(np.float64) / r
  return np.cos(ph).astype(np.float32), (-np.sin(ph)).astype(np.float32)

D1C, D1S = _dft_tables(R1)
D2C, D2S = _dft_tables(R2)
D3C, D3S = _dft_tables(R3)

_n2 = np.arange(R2)
_a = np.arange(R1)
_ph = 2.0 * np.pi * ((R3 * np.outer(_a, _n2)) % N).astype(np.float64) / N
T1C = np.cos(_ph).astype(np.float32)
T1S = (-np.sin(_ph)).astype(np.float32)

_c = np.arange(R3)
_ph = 2.0 * np.pi * (np.outer(_a, _c) % N).astype(np.float64) / N
TAC = np.cos(_ph).astype(np.float32)
TAS = (-np.sin(_ph)).astype(np.float32)
_b = np.arange(R2)
_ph = 2.0 * np.pi * (np.outer(_b, _c) % N23).astype(np.float64) / N23
TBC = np.cos(_ph).astype(np.float32)
TBS = (-np.sin(_ph)).astype(np.float32)


S1_COLS = 2048
S1_GRID = N23 // S1_COLS
AB_CHUNK = 8
S23_ROWS = AB_CHUNK * R2
S23_GRID = R1 // AB_CHUNK


def _fft1_body(d1c_ref, d1s_ref, x_ref, yre_ref, yim_ref):
  xb = x_ref[...]
  yre_ref[...] = jnp.dot(d1c_ref[...], xb,
                         preferred_element_type=jnp.float32, precision=PREC)
  yim_ref[...] = jnp.dot(d1s_ref[...], xb,
                         preferred_element_type=jnp.float32, precision=PREC)


def _fft2_body(d2c_ref, d2s_ref, t1c_ref, t1s_ref, yre_ref, yim_ref,
               zre_ref, zim_ref):
  d2c = d2c_ref[...]
  d2s = d2s_ref[...]
  for j in range(AB_CHUNK):
    yre = yre_ref[pl.ds(j * R2, R2), :]
    yim = yim_ref[pl.ds(j * R2, R2), :]
    t1c = t1c_ref[pl.ds(j, 1), :]
    t1s = t1s_ref[pl.ds(j, 1), :]
    d2ca = d2c * t1c - d2s * t1s
    d2sa = d2c * t1s + d2s * t1c
    zre_ref[pl.ds(j * R2, R2), :] = (
        jnp.dot(d2ca, yre, preferred_element_type=jnp.float32, precision=PREC)
        - jnp.dot(d2sa, yim, preferred_element_type=jnp.float32, precision=PREC))
    zim_ref[pl.ds(j * R2, R2), :] = (
        jnp.dot(d2ca, yim, preferred_element_type=jnp.float32, precision=PREC)
        + jnp.dot(d2sa, yre, preferred_element_type=jnp.float32, precision=PREC))


def _fft3_body(d3c_ref, d3s_ref, tbc_ref, tbs_ref, tac_ref, tas_ref,
               zre_ref, zim_ref, ore_ref, oim_ref):
  d3c = d3c_ref[...]
  d3s = d3s_ref[...]
  tbc = tbc_ref[...]
  tbs = tbs_ref[...]
  for j in range(AB_CHUNK):
    zre = zre_ref[pl.ds(j * R2, R2), :]
    zim = zim_ref[pl.ds(j * R2, R2), :]
    tac = tac_ref[pl.ds(j, 1), :]
    tas = tas_ref[pl.ds(j, 1), :]
    t2c = tac * tbc - tas * tbs
    t2s = tac * tbs + tas * tbc
    ztre = zre * t2c - zim * t2s
    ztim = zre * t2s + zim * t2c
    ore_ref[pl.ds(j * R2, R2), :] = (
        jnp.dot(ztre, d3c, preferred_element_type=jnp.float32, precision=PREC)
        - jnp.dot(ztim, d3s, preferred_element_type=jnp.float32, precision=PREC))
    oim_ref[pl.ds(j * R2, R2), :] = (
        jnp.dot(ztre, d3s, preferred_element_type=jnp.float32, precision=PREC)
        + jnp.dot(ztim, d3c, preferred_element_type=jnp.float32, precision=PREC))


def _fft(x):
  xm = x.reshape(R1, N23)
  full = pl.BlockSpec((R1, R1), lambda i: (0, 0))
  yre, yim = pl.pallas_call(
      _fft1_body,
      grid=(S1_GRID,),
      in_specs=[full, full, pl.BlockSpec((R1, S1_COLS), lambda i: (0, i))],
      out_specs=[pl.BlockSpec((R1, S1_COLS), lambda i: (0, i))] * 2,
      out_shape=[jax.ShapeDtypeStruct((R1, N23), jnp.float32)] * 2,
  )(jnp.asarray(D1C), jnp.asarray(D1S), xm)

  yre = yre.reshape(R1 * R2, R3)
  yim = yim.reshape(R1 * R2, R3)
  rows = pl.BlockSpec((S23_ROWS, R3), lambda i: (i, 0))
  d2full = pl.BlockSpec((R2, R2), lambda i: (0, 0))
  t1spec = pl.BlockSpec((AB_CHUNK, R2), lambda i: (i, 0))
  zre, zim = pl.pallas_call(
      _fft2_body,
      grid=(S23_GRID,),
      in_specs=[d2full, d2full, t1spec, t1spec, rows, rows],
      out_specs=[rows, rows],
      out_shape=[jax.ShapeDtypeStruct((R1 * R2, R3), jnp.float32)] * 2,
  )(jnp.asarray(D2C), jnp.asarray(D2S), jnp.asarray(T1C), jnp.asarray(T1S),
    yre, yim)

  d3full = pl.BlockSpec((R3, R3), lambda i: (0, 0))
  tbspec = pl.BlockSpec((R2, R3), lambda i: (0, 0))
  taspec = pl.BlockSpec((AB_CHUNK, R3), lambda i: (i, 0))
  ore, oim = pl.pallas_call(
      _fft3_body,
      grid=(S23_GRID,),
      in_specs=[d3full, d3full, tbspec, tbspec, taspec, taspec, rows, rows],
      out_specs=[rows, rows],
      out_shape=[jax.ShapeDtypeStruct((R1 * R2, R3), jnp.float32)] * 2,
  )(jnp.asarray(D3C), jnp.asarray(D3S), jnp.asarray(TBC), jnp.asarray(TBS),
    jnp.asarray(TAC), jnp.asarray(TAS), zre, zim)
  return ore, oim


NW = 32
TOT_ROWS = R1 * R2
ROWS_PER_W = TOT_ROWS // NW
R_CHUNK = 128
N_DMA = ROWS_PER_W // R_CHUNK
VALID_COLS = 64
SUB_ROWS = 8
N_SUB = R_CHUNK // SUB_ROWS
VEC = 16
VPR = VALID_COLS // VEC
N_VEC = SUB_ROWS * VPR


def _merge_pool(vm, vi, vre, vim, poolm, pooli, poolre, poolim, lane):
  nk, si, sre, sim = lax.sort((-vm, vi, vre, vim), num_keys=1)
  sm = -nk
  sel = lane < 8
  cm = jnp.where(sel, poolm, lax.rev(sm, (0,)))
  ci = jnp.where(sel, pooli, lax.rev(si, (0,)))
  cre = jnp.where(sel, poolre, lax.rev(sre, (0,)))
  cim = jnp.where(sel, poolim, lax.rev(sim, (0,)))
  nk2, pi2, pre2, pim2 = lax.sort((-cm, ci, cre, cim), num_keys=1)
  pm2 = -nk2
  new_t = jnp.min(jnp.where(lane < 5, pm2, jnp.float32(3e38)))
  return pm2, pi2, pre2, pim2, new_t


def _sc_topk_body(re_hbm, im_hbm, outm, outi, outre, outim,
                  re_buf, im_buf, stg_m, stg_i, stg_re, stg_im):
  wid = lax.axis_index("s") * 2 + lax.axis_index("c")
  row_base = wid * ROWS_PER_W
  lane = lax.iota(jnp.int32, 16)
  lane_k = lane << 16

  state0 = (
      jnp.full((VEC,), -1.0, jnp.float32),
      jnp.zeros((VEC,), jnp.int32),
      jnp.zeros((VEC,), jnp.float32),
      jnp.zeros((VEC,), jnp.float32),
      jnp.float32(-1.0),
  )

  def process_sub(sc, chunk_row0, state):
    r0 = sc * SUB_ROWS

    def maxbody(v, acc):
      r = r0 + (v >> 2)
      cc = (v & 3) * VEC
      rr = re_buf[r, pl.ds(cc, VEC)]
      ii = im_buf[r, pl.ds(cc, VEC)]
      return jnp.maximum(acc, rr * rr + ii * ii)

    m_acc = lax.fori_loop(0, N_VEC, maxbody, jnp.full((VEC,), -2.0, jnp.float32))
    sub_max = jnp.max(m_acc)

    def rescan(st):
      def body(v, st2):
        r = r0 + (v >> 2)
        cc = (v & 3) * VEC
        rr = re_buf[r, pl.ds(cc, VEC)]
        ii = im_buf[r, pl.ds(cc, VEC)]
        vm = rr * rr + ii * ii
        vmax = jnp.max(vm)

        def do_merge(st3):
          pm, pi, pre, pim, _t = st3
          grow = chunk_row0 + r
          a = grow >> 8
          b = grow & 255
          base_k = a + (b << 8) + (cc << 16)
          vi = lane_k + base_k
          return _merge_pool(vm, vi, rr, ii, pm, pi, pre, pim, lane)

        return lax.cond(vmax > st2[4], do_merge, lambda s: s, st2)

      return lax.fori_loop(0, N_VEC, body, st)

    return lax.cond(sub_max > state[4], rescan, lambda s: s, state)

  def dma_step(s, state):
    row0 = row_base + s * R_CHUNK
    pltpu.sync_copy(re_hbm.at[pl.ds(row0, R_CHUNK), pl.ds(0, VALID_COLS)],
                    re_buf)
    pltpu.sync_copy(im_hbm.at[pl.ds(row0, R_CHUNK), pl.ds(0, VALID_COLS)],
                    im_buf)

    @pl.when(jnp.logical_and(wid == 0, s == 0))
    def _zero_dc():
      rr = re_buf[0, pl.ds(0, VEC)]
      ii = im_buf[0, pl.ds(0, VEC)]
      re_buf[0, pl.ds(0, VEC)] = jnp.where(lane == 0, jnp.float32(0.0), rr)
      im_buf[0, pl.ds(0, VEC)] = jnp.where(lane == 0, jnp.float32(0.0), ii)

    def sub(c, st):
      return process_sub(c, row0, st)

    return lax.fori_loop(0, N_SUB, sub, state)

  state = lax.fori_loop(0, N_DMA, dma_step, state0)

  def nyq_merge(st):
    pltpu.sync_copy(re_hbm.at[0, pl.ds(VALID_COLS, VEC)], stg_re)
    pltpu.sync_copy(im_hbm.at[0, pl.ds(VALID_COLS, VEC)], stg_im)
    rr = stg_re[...]
    ii = stg_im[...]
    vm = jnp.where(lane == 0, rr * rr + ii * ii, jnp.float32(-1.0))
    vi = jnp.full((VEC,), NHALF, jnp.int32)
    pm, pi, pre, pim, _t = st
    return _merge_pool(vm, vi, rr, ii, pm, pi, pre, pim, lane)

  state = lax.cond(wid == 0, nyq_merge, lambda s: s, state)
  poolm, pooli, poolre, poolim, _t = state

  stg_m[...] = poolm
  stg_i[...] = pooli
  stg_re[...] = poolre
  stg_im[...] = poolim
  pltpu.sync_copy(stg_m, outm.at[pl.ds(wid * VEC, VEC)])
  pltpu.sync_copy(stg_i, outi.at[pl.ds(wid * VEC, VEC)])
  pltpu.sync_copy(stg_re, outre.at[pl.ds(wid * VEC, VEC)])
  pltpu.sync_copy(stg_im, outim.at[pl.ds(wid * VEC, VEC)])


@functools.cache
def _sc_topk():
  return pl.kernel(
      _sc_topk_body,
      out_type=[
          jax.ShapeDtypeStruct((NW * VEC,), jnp.float32),
          jax.ShapeDtypeStruct((NW * VEC,), jnp.int32),
          jax.ShapeDtypeStruct((NW * VEC,), jnp.float32),
          jax.ShapeDtypeStruct((NW * VEC,), jnp.float32),
      ],
      mesh=plsc.VectorSubcoreMesh(core_axis_name="c", subcore_axis_name="s"),
      compiler_params=pltpu.CompilerParams(needs_layout_passes=False,
                                           use_tc_tiling_on_sc=False),
      scratch_types=[
          pltpu.VMEM((R_CHUNK, VALID_COLS), jnp.float32),
          pltpu.VMEM((R_CHUNK, VALID_COLS), jnp.float32),
          pltpu.VMEM((VEC,), jnp.float32),
          pltpu.VMEM((VEC,), jnp.int32),
          pltpu.VMEM((VEC,), jnp.float32),
          pltpu.VMEM((VEC,), jnp.float32),
      ],
  )


ROWS = 8192
COLS = 1024
BLK_R = 256
GRID = ROWS // BLK_R


def _tc_synth_body(candm_ref, candi_ref, candre_ref, candim_ref, x_ref,
                   season_ref, trend_ref):
  m = candm_ref[...]
  ci = candi_ref[...]
  cre = candre_ref[...]
  cim = candim_ref[...]
  fi = (lax.broadcasted_iota(jnp.int32, m.shape, 0) * m.shape[1]
        + lax.broadcasted_iota(jnp.int32, m.shape, 1))
  avail = fi >= 0
  vals = []
  for _ in range(5):
    cur = jnp.where(avail, m, jnp.float32(-3.0))
    mk = jnp.max(cur)
    pick = jnp.min(jnp.where(cur == mk, fi, jnp.int32(1 << 30)))
    sel = fi == pick
    jk = jnp.sum(jnp.where(sel, ci, 0))
    rek = jnp.sum(jnp.where(sel, cre, jnp.float32(0.0)))
    imk = jnp.sum(jnp.where(sel, cim, jnp.float32(0.0)))
    avail = jnp.logical_and(avail, jnp.logical_not(sel))
    vals.append((mk, jk, rek, imk))
  thresh2 = vals[4][0]

  i = pl.program_id(0)
  n1 = i * BLK_R + lax.broadcasted_iota(jnp.int32, (BLK_R, 1), 0)
  n2 = lax.broadcasted_iota(jnp.int32, (1, COLS), 1)
  season = jnp.zeros((BLK_R, COLS), jnp.float32)
  for k in range(5):
    mk, jk, rek, imk = vals[k]
    alive = mk > thresh2
    is_nyq = jk == NHALF
    w = jnp.where(is_nyq, jnp.float32(1.0), jnp.float32(2.0)) * jnp.float32(1.0 / N)
    a = jnp.where(alive, w * rek, jnp.float32(0.0))
    b = jnp.where(jnp.logical_and(alive, jnp.logical_not(is_nyq)),
                  -w * imk, jnp.float32(0.0))
    m1 = (jk * (n1 * COLS)) & PHASE_MASK
    m2 = (jk * n2) & PHASE_MASK
    th1 = m1.astype(jnp.float32) * jnp.float32(TWO_PI_OVER_N)
    th2 = m2.astype(jnp.float32) * jnp.float32(TWO_PI_OVER_N)
    c1 = jnp.cos(th1)
    s1 = jnp.sin(th1)
    c2 = jnp.cos(th2)
    s2 = jnp.sin(th2)
    p = a * c1 + b * s1
    q = b * c1 - a * s1
    season = season + p * c2 + q * s2
  xv = x_ref[...]
  season_ref[...] = season
  trend_ref[...] = xv - season


def _tc_synth(candm, candi, candre, candim, x2):
  cand_spec = pl.BlockSpec((4, 128), lambda i: (0, 0))
  return pl.pallas_call(
      _tc_synth_body,
      grid=(GRID,),
      in_specs=[cand_spec, cand_spec, cand_spec, cand_spec,
                pl.BlockSpec((BLK_R, COLS), lambda i: (i, 0))],
      out_specs=[pl.BlockSpec((BLK_R, COLS), lambda i: (i, 0)),
                 pl.BlockSpec((BLK_R, COLS), lambda i: (i, 0))],
      out_shape=[jax.ShapeDtypeStruct((ROWS, COLS), jnp.float32),
                 jax.ShapeDtypeStruct((ROWS, COLS), jnp.float32)],
  )(candm, candi, candre, candim, x2)


def kernel(x):
  ore, oim = _fft(x)
  candm, candi, candre, candim = _sc_topk()(ore, oim)
  x2 = x.reshape(ROWS, COLS)
  season, trend = _tc_synth(candm.reshape(4, 128), candi.reshape(4, 128),
                            candre.reshape(4, 128), candim.reshape(4, 128), x2)
  return season.reshape(-1), trend.reshape(-1)


# speedup vs baseline: 13.1267x; 1.1927x over previous
import functools

import numpy as np

import jax
import jax.numpy as jnp
from jax import lax
from jax.experimental import pallas as pl
from jax.experimental.pallas import tpu as pltpu
from jax.experimental.pallas import tpu_sc as plsc

N = 8388608
NHALF = N // 2
PHASE_MASK = N - 1
TWO_PI_OVER_N = 2.0 * np.pi / N

R1 = 256
R2 = 256
R3 = 128
N23 = R2 * R3

PREC = jax.lax.Precision.HIGHEST


def _dft_tables(r):
  k = np.arange(r)
  ph = 2.0 * np.pi * (np.outer(k, k) % r).astype(np.float64) / r
  return np.cos(ph).astype(np.float32), (-np.sin(ph)).astype(np.float32)

D1C, D1S = _dft_tables(R1)
D2C, D2S = _dft_tables(R2)
D3C, D3S = _dft_tables(R3)

_n2 = np.arange(R2)
_a = np.arange(R1)
_ph = 2.0 * np.pi * ((R3 * np.outer(_a, _n2)) % N).astype(np.float64) / N
T1C = np.cos(_ph).astype(np.float32)
T1S = (-np.sin(_ph)).astype(np.float32)

_c = np.arange(R3)
_ph = 2.0 * np.pi * (np.outer(_a, _c) % N).astype(np.float64) / N
TAC = np.cos(_ph).astype(np.float32)
TAS = (-np.sin(_ph)).astype(np.float32)
_b = np.arange(R2)
_ph = 2.0 * np.pi * (np.outer(_b, _c) % N23).astype(np.float64) / N23
TBC = np.cos(_ph).astype(np.float32)
TBS = (-np.sin(_ph)).astype(np.float32)


S1_COLS = 2048
S1_GRID = N23 // S1_COLS
AB_CHUNK = 8
S23_ROWS = AB_CHUNK * R2
S23_GRID = R1 // AB_CHUNK


N2_CHUNK = S1_COLS // R3


def _fft1_body(d1c_ref, d1s_ref, x_ref, yre_ref, yim_ref):
  xb = x_ref[...]
  yre = jnp.dot(d1c_ref[...], xb,
                preferred_element_type=jnp.float32, precision=PREC)
  yim = jnp.dot(d1s_ref[...], xb,
                preferred_element_type=jnp.float32, precision=PREC)
  for j in range(N2_CHUNK):
    yre_ref[:, j, :] = yre[:, j * R3:(j + 1) * R3]
    yim_ref[:, j, :] = yim[:, j * R3:(j + 1) * R3]


def _dot(a, b):
  return jnp.dot(a, b, preferred_element_type=jnp.float32, precision=PREC)


def _fft23_body(d2c_ref, d2s_ref, t1c_ref, t1s_ref, d3c_ref, d3s_ref,
                d3cs_ref, tbc_ref, tbs_ref, tac_ref, tas_ref,
                yre_ref, yim_ref, ore_ref, oim_ref):
  d2c = d2c_ref[...]
  d2s = d2s_ref[...]
  d3c = d3c_ref[...]
  d3s = d3s_ref[...]
  d3cs = d3cs_ref[...]
  tbc = tbc_ref[...]
  tbs = tbs_ref[...]
  for j in range(AB_CHUNK):
    yre = yre_ref[j]
    yim = yim_ref[j]
    t1c = t1c_ref[pl.ds(j, 1), :]
    t1s = t1s_ref[pl.ds(j, 1), :]
    d2ca = d2c * t1c - d2s * t1s
    d2sa = d2c * t1s + d2s * t1c
    m1 = _dot(d2ca, yre)
    m2 = _dot(d2sa, yim)
    m3 = _dot(d2ca + d2sa, yre + yim)
    zre = m1 - m2
    zim = m3 - m1 - m2
    tac = tac_ref[pl.ds(j, 1), :]
    tas = tas_ref[pl.ds(j, 1), :]
    t2c = tac * tbc - tas * tbs
    t2s = tac * tbs + tas * tbc
    ztre = zre * t2c - zim * t2s
    ztim = zre * t2s + zim * t2c
    k1 = _dot(ztre, d3c)
    k2 = _dot(ztim, d3s)
    k3 = _dot(ztre + ztim, d3cs)
    ore_ref[j] = k1 - k2
    oim_ref[j] = k3 - k1 - k2


def _fft(x):
  xm = x.reshape(R1, N23)
  full = pl.BlockSpec((R1, R1), lambda i: (0, 0))
  y3 = pl.BlockSpec((R1, N2_CHUNK, R3), lambda i: (0, i, 0))
  yre, yim = pl.pallas_call(
      _fft1_body,
      grid=(S1_GRID,),
      in_specs=[full, full, pl.BlockSpec((R1, S1_COLS), lambda i: (0, i))],
      out_specs=[y3, y3],
      out_shape=[jax.ShapeDtypeStruct((R1, R2, R3), jnp.float32)] * 2,
  )(jnp.asarray(D1C), jnp.asarray(D1S), xm)

  rows3 = pl.BlockSpec((AB_CHUNK, R2, R3), lambda i: (i, 0, 0))
  d2full = pl.BlockSpec((R2, R2), lambda i: (0, 0))
  t1spec = pl.BlockSpec((AB_CHUNK, R2), lambda i: (i, 0))
  d3full = pl.BlockSpec((R3, R3), lambda i: (0, 0))
  tbspec = pl.BlockSpec((R2, R3), lambda i: (0, 0))
  taspec = pl.BlockSpec((AB_CHUNK, R3), lambda i: (i, 0))
  ore, oim = pl.pallas_call(
      _fft23_body,
      grid=(S23_GRID,),
      in_specs=[d2full, d2full, t1spec, t1spec, d3full, d3full, d3full,
                tbspec, tbspec, taspec, taspec, rows3, rows3],
      out_specs=[rows3, rows3],
      out_shape=[jax.ShapeDtypeStruct((R1, R2, R3), jnp.float32)] * 2,
  )(jnp.asarray(D2C), jnp.asarray(D2S), jnp.asarray(T1C), jnp.asarray(T1S),
    jnp.asarray(D3C), jnp.asarray(D3S), jnp.asarray(D3C + D3S),
    jnp.asarray(TBC), jnp.asarray(TBS), jnp.asarray(TAC), jnp.asarray(TAS),
    yre, yim)
  return ore.reshape(R1 * R2, R3), oim.reshape(R1 * R2, R3)


NW = 32
TOT_ROWS = R1 * R2
ROWS_PER_W = TOT_ROWS // NW
R_CHUNK = 128
N_DMA = ROWS_PER_W // R_CHUNK
VALID_COLS = 64
SUB_ROWS = 8
N_SUB = R_CHUNK // SUB_ROWS
VEC = 16
VPR = VALID_COLS // VEC
N_VEC = SUB_ROWS * VPR


def _merge_pool(vm, vi, vre, vim, poolm, pooli, poolre, poolim, lane):
  nk, si, sre, sim = lax.sort((-vm, vi, vre, vim), num_keys=1)
  sm = -nk
  sel = lane < 8
  cm = jnp.where(sel, poolm, lax.rev(sm, (0,)))
  ci = jnp.where(sel, pooli, lax.rev(si, (0,)))
  cre = jnp.where(sel, poolre, lax.rev(sre, (0,)))
  cim = jnp.where(sel, poolim, lax.rev(sim, (0,)))
  nk2, pi2, pre2, pim2 = lax.sort((-cm, ci, cre, cim), num_keys=1)
  pm2 = -nk2
  new_t = jnp.min(jnp.where(lane < 5, pm2, jnp.float32(3e38)))
  return pm2, pi2, pre2, pim2, new_t


def _sc_topk_body(re_hbm, im_hbm, outm, outi, outre, outim,
                  re_buf, im_buf, stg_m, stg_i, stg_re, stg_im):
  wid = lax.axis_index("s") * 2 + lax.axis_index("c")
  row_base = wid * ROWS_PER_W
  lane = lax.iota(jnp.int32, 16)
  lane_k = lane << 16

  state0 = (
      jnp.full((VEC,), -1.0, jnp.float32),
      jnp.zeros((VEC,), jnp.int32),
      jnp.zeros((VEC,), jnp.float32),
      jnp.zeros((VEC,), jnp.float32),
      jnp.float32(-1.0),
  )

  def process_sub(sc, chunk_row0, state):
    r0 = sc * SUB_ROWS

    def maxbody(v, acc):
      r = r0 + (v >> 2)
      cc = (v & 3) * VEC
      rr = re_buf[r, pl.ds(cc, VEC)]
      ii = im_buf[r, pl.ds(cc, VEC)]
      return jnp.maximum(acc, rr * rr + ii * ii)

    m_acc = lax.fori_loop(0, N_VEC, maxbody, jnp.full((VEC,), -2.0, jnp.float32))
    sub_max = jnp.max(m_acc)

    def rescan(st):
      def body(v, st2):
        r = r0 + (v >> 2)
        cc = (v & 3) * VEC
        rr = re_buf[r, pl.ds(cc, VEC)]
        ii = im_buf[r, pl.ds(cc, VEC)]
        vm = rr * rr + ii * ii
        vmax = jnp.max(vm)

        def do_merge(st3):
          pm, pi, pre, pim, _t = st3
          grow = chunk_row0 + r
          a = grow >> 8
          b = grow & 255
          base_k = a + (b << 8) + (cc << 16)
          vi = lane_k + base_k
          return _merge_pool(vm, vi, rr, ii, pm, pi, pre, pim, lane)

        return lax.cond(vmax > st2[4], do_merge, lambda s: s, st2)

      return lax.fori_loop(0, N_VEC, body, st)

    return lax.cond(sub_max > state[4], rescan, lambda s: s, state)

  def dma_step(s, state):
    row0 = row_base + s * R_CHUNK
    pltpu.sync_copy(re_hbm.at[pl.ds(row0, R_CHUNK), pl.ds(0, VALID_COLS)],
                    re_buf)
    pltpu.sync_copy(im_hbm.at[pl.ds(row0, R_CHUNK), pl.ds(0, VALID_COLS)],
                    im_buf)

    @pl.when(jnp.logical_and(wid == 0, s == 0))
    def _zero_dc():
      rr = re_buf[0, pl.ds(0, VEC)]
      ii = im_buf[0, pl.ds(0, VEC)]
      re_buf[0, pl.ds(0, VEC)] = jnp.where(lane == 0, jnp.float32(0.0), rr)
      im_buf[0, pl.ds(0, VEC)] = jnp.where(lane == 0, jnp.float32(0.0), ii)

    def sub(c, st):
      return process_sub(c, row0, st)

    return lax.fori_loop(0, N_SUB, sub, state)

  state = lax.fori_loop(0, N_DMA, dma_step, state0)

  def nyq_merge(st):
    pltpu.sync_copy(re_hbm.at[0, pl.ds(VALID_COLS, VEC)], stg_re)
    pltpu.sync_copy(im_hbm.at[0, pl.ds(VALID_COLS, VEC)], stg_im)
    rr = stg_re[...]
    ii = stg_im[...]
    vm = jnp.where(lane == 0, rr * rr + ii * ii, jnp.float32(-1.0))
    vi = jnp.full((VEC,), NHALF, jnp.int32)
    pm, pi, pre, pim, _t = st
    return _merge_pool(vm, vi, rr, ii, pm, pi, pre, pim, lane)

  state = lax.cond(wid == 0, nyq_merge, lambda s: s, state)
  poolm, pooli, poolre, poolim, _t = state

  stg_m[...] = poolm
  stg_i[...] = pooli
  stg_re[...] = poolre
  stg_im[...] = poolim
  pltpu.sync_copy(stg_m, outm.at[pl.ds(wid * VEC, VEC)])
  pltpu.sync_copy(stg_i, outi.at[pl.ds(wid * VEC, VEC)])
  pltpu.sync_copy(stg_re, outre.at[pl.ds(wid * VEC, VEC)])
  pltpu.sync_copy(stg_im, outim.at[pl.ds(wid * VEC, VEC)])


@functools.cache
def _sc_topk():
  return pl.kernel(
      _sc_topk_body,
      out_type=[
          jax.ShapeDtypeStruct((NW * VEC,), jnp.float32),
          jax.ShapeDtypeStruct((NW * VEC,), jnp.int32),
          jax.ShapeDtypeStruct((NW * VEC,), jnp.float32),
          jax.ShapeDtypeStruct((NW * VEC,), jnp.float32),
      ],
      mesh=plsc.VectorSubcoreMesh(core_axis_name="c", subcore_axis_name="s"),
      compiler_params=pltpu.CompilerParams(needs_layout_passes=False,
                                           use_tc_tiling_on_sc=False),
      scratch_types=[
          pltpu.VMEM((R_CHUNK, VALID_COLS), jnp.float32),
          pltpu.VMEM((R_CHUNK, VALID_COLS), jnp.float32),
          pltpu.VMEM((VEC,), jnp.float32),
          pltpu.VMEM((VEC,), jnp.int32),
          pltpu.VMEM((VEC,), jnp.float32),
          pltpu.VMEM((VEC,), jnp.float32),
      ],
  )


ROWS = 8192
COLS = 1024
BLK_R = 256
GRID = ROWS // BLK_R


def _tc_synth_body(candm_ref, candi_ref, candre_ref, candim_ref, x_ref,
                   season_ref, trend_ref, uv_ref, jk_smem):
  i = pl.program_id(0)

  @pl.when(i == 0)
  def _init():
    m = candm_ref[...]
    ci = candi_ref[...]
    cre = candre_ref[...]
    cim = candim_ref[...]
    fi = (lax.broadcasted_iota(jnp.int32, m.shape, 0) * m.shape[1]
          + lax.broadcasted_iota(jnp.int32, m.shape, 1))
    avail = fi >= 0
    vals = []
    for _ in range(5):
      cur = jnp.where(avail, m, jnp.float32(-3.0))
      mk = jnp.max(cur)
      pick = jnp.min(jnp.where(cur == mk, fi, jnp.int32(1 << 30)))
      sel = fi == pick
      jk = jnp.sum(jnp.where(sel, ci, 0))
      rek = jnp.sum(jnp.where(sel, cre, jnp.float32(0.0)))
      imk = jnp.sum(jnp.where(sel, cim, jnp.float32(0.0)))
      avail = jnp.logical_and(avail, jnp.logical_not(sel))
      vals.append((mk, jk, rek, imk))
    thresh2 = vals[4][0]
    n2 = lax.broadcasted_iota(jnp.int32, (1, COLS), 1)
    zero_row = jnp.zeros((1, COLS), jnp.float32)
    for k in range(5):
      mk, jk, rek, imk = vals[k]
      alive = mk > thresh2
      is_nyq = jk == NHALF
      w = (jnp.where(is_nyq, jnp.float32(1.0), jnp.float32(2.0))
           * jnp.float32(1.0 / N))
      a = jnp.where(alive, w * rek, jnp.float32(0.0))
      b = jnp.where(jnp.logical_and(alive, jnp.logical_not(is_nyq)),
                    -w * imk, jnp.float32(0.0))
      m2 = (jk * n2) & PHASE_MASK
      th2 = m2.astype(jnp.float32) * jnp.float32(TWO_PI_OVER_N)
      c2 = jnp.cos(th2)
      s2 = jnp.sin(th2)
      uv_ref[pl.ds(k, 1), :] = a * c2 + b * s2
      uv_ref[pl.ds(8 + k, 1), :] = b * c2 - a * s2
      jk_smem[k] = jk
    for k in range(5, 8):
      uv_ref[pl.ds(k, 1), :] = zero_row
      uv_ref[pl.ds(8 + k, 1), :] = zero_row
      jk_smem[k] = 0

  n1k = (i * BLK_R + lax.broadcasted_iota(jnp.int32, (BLK_R, 1), 0)) * COLS
  jvec = jnp.concatenate(
      [lax.broadcast_in_dim(jk_smem[k], (1, 1), ()) for k in range(8)], axis=1)
  m1 = (jvec * n1k) & PHASE_MASK
  th1 = m1.astype(jnp.float32) * jnp.float32(TWO_PI_OVER_N)
  cs1 = jnp.concatenate([jnp.cos(th1), jnp.sin(th1)], axis=1)
  season = jnp.dot(cs1, uv_ref[...],
                   preferred_element_type=jnp.float32, precision=PREC)
  season_ref[...] = season
  trend_ref[...] = x_ref[...] - season


def _tc_synth(candm, candi, candre, candim, x2):
  cand_spec = pl.BlockSpec((4, 128), lambda i: (0, 0))
  return pl.pallas_call(
      _tc_synth_body,
      grid=(GRID,),
      in_specs=[cand_spec, cand_spec, cand_spec, cand_spec,
                pl.BlockSpec((BLK_R, COLS), lambda i: (i, 0))],
      out_specs=[pl.BlockSpec((BLK_R, COLS), lambda i: (i, 0)),
                 pl.BlockSpec((BLK_R, COLS), lambda i: (i, 0))],
      out_shape=[jax.ShapeDtypeStruct((ROWS, COLS), jnp.float32),
                 jax.ShapeDtypeStruct((ROWS, COLS), jnp.float32)],
      scratch_shapes=[pltpu.VMEM((16, COLS), jnp.float32),
                      pltpu.SMEM((8,), jnp.int32)],
  )(candm, candi, candre, candim, x2)


def kernel(x):
  ore, oim = _fft(x)
  candm, candi, candre, candim = _sc_topk()(ore, oim)
  x2 = x.reshape(ROWS, COLS)
  season, trend = _tc_synth(candm.reshape(4, 128), candi.reshape(4, 128),
                            candre.reshape(4, 128), candim.reshape(4, 128), x2)
  return season.reshape(-1), trend.reshape(-1)


# speedup vs baseline: 18.9484x; 1.4435x over previous
import functools

import numpy as np

import jax
import jax.numpy as jnp
from jax import lax
from jax.experimental import pallas as pl
from jax.experimental.pallas import tpu as pltpu
from jax.experimental.pallas import tpu_sc as plsc

N = 8388608
NHALF = N // 2
PHASE_MASK = N - 1
TWO_PI_OVER_N = 2.0 * np.pi / N

R1 = 256
R2 = 256
R3 = 128
N23 = R2 * R3

PREC = jax.lax.Precision.HIGHEST


def _dft_tables(r):
  k = np.arange(r)
  ph = 2.0 * np.pi * (np.outer(k, k) % r).astype(np.float64) / r
  return np.cos(ph).astype(np.float32), (-np.sin(ph)).astype(np.float32)

D1C, D1S = _dft_tables(R1)
D2C, D2S = _dft_tables(R2)
D3C, D3S = _dft_tables(R3)

_n2 = np.arange(R2)
_a = np.arange(R1)
_ph = 2.0 * np.pi * ((R3 * np.outer(_a, _n2)) % N).astype(np.float64) / N
T1C = np.cos(_ph).astype(np.float32)
T1S = (-np.sin(_ph)).astype(np.float32)

_c = np.arange(R3)
_ph = 2.0 * np.pi * (np.outer(_a, _c) % N).astype(np.float64) / N
TAC = np.cos(_ph).astype(np.float32)
TAS = (-np.sin(_ph)).astype(np.float32)
_b = np.arange(R2)
_ph = 2.0 * np.pi * (np.outer(_b, _c) % N23).astype(np.float64) / N23
TBC = np.cos(_ph).astype(np.float32)
TBS = (-np.sin(_ph)).astype(np.float32)


S1_COLS = 2048
S1_GRID = N23 // S1_COLS
AB_CHUNK = 8
S23_ROWS = AB_CHUNK * R2
S23_GRID = R1 // AB_CHUNK


N2_CHUNK = S1_COLS // R3


def _fft1_body(d1c_ref, d1s_ref, x_ref, yre_ref, yim_ref):
  xs = _split(x_ref[...])
  yre = _dot3s(_split(d1c_ref[...]), xs)
  yim = _dot3s(_split(d1s_ref[...]), xs)
  for j in range(N2_CHUNK):
    yre_ref[:, j, :] = yre[:, j * R3:(j + 1) * R3]
    yim_ref[:, j, :] = yim[:, j * R3:(j + 1) * R3]


def _split(a):
  ah = a.astype(jnp.bfloat16)
  al = (a - ah.astype(jnp.float32)).astype(jnp.bfloat16)
  return ah, al


def _dot3s(a_split, b_split):
  ah, al = a_split
  bh, bl = b_split
  d = functools.partial(jnp.dot, preferred_element_type=jnp.float32)
  return d(ah, bh) + d(ah, bl) + d(al, bh)


def _dot3(a, b):
  return _dot3s(_split(a), _split(b))


def _fft23_body(d2c_ref, d2s_ref, t1c_ref, t1s_ref, d3c_ref, d3s_ref,
                d3cs_ref, tbc_ref, tbs_ref, tac_ref, tas_ref,
                yre_ref, yim_ref, ore_ref, oim_ref):
  d2c = d2c_ref[...]
  d2s = d2s_ref[...]
  d3c_s = _split(d3c_ref[...])
  d3s_s = _split(d3s_ref[...])
  d3cs_s = _split(d3cs_ref[...])
  tbc = tbc_ref[...]
  tbs = tbs_ref[...]
  for j in range(AB_CHUNK):
    yre = yre_ref[j]
    yim = yim_ref[j]
    t1c = t1c_ref[pl.ds(j, 1), :]
    t1s = t1s_ref[pl.ds(j, 1), :]
    d2ca = d2c * t1c - d2s * t1s
    d2sa = d2c * t1s + d2s * t1c
    m1 = _dot3(d2ca, yre)
    m2 = _dot3(d2sa, yim)
    m3 = _dot3(d2ca + d2sa, yre + yim)
    zre = m1 - m2
    zim = m3 - m1 - m2
    tac = tac_ref[pl.ds(j, 1), :]
    tas = tas_ref[pl.ds(j, 1), :]
    t2c = tac * tbc - tas * tbs
    t2s = tac * tbs + tas * tbc
    ztre = zre * t2c - zim * t2s
    ztim = zre * t2s + zim * t2c
    k1 = _dot3s(_split(ztre), d3c_s)
    k2 = _dot3s(_split(ztim), d3s_s)
    k3 = _dot3s(_split(ztre + ztim), d3cs_s)
    ore_ref[j] = k1 - k2
    oim_ref[j] = k3 - k1 - k2


def _fft(x):
  xm = x.reshape(R1, N23)
  full = pl.BlockSpec((R1, R1), lambda i: (0, 0))
  y3 = pl.BlockSpec((R1, N2_CHUNK, R3), lambda i: (0, i, 0))
  yre, yim = pl.pallas_call(
      _fft1_body,
      grid=(S1_GRID,),
      in_specs=[full, full, pl.BlockSpec((R1, S1_COLS), lambda i: (0, i))],
      out_specs=[y3, y3],
      out_shape=[jax.ShapeDtypeStruct((R1, R2, R3), jnp.float32)] * 2,
  )(jnp.asarray(D1C), jnp.asarray(D1S), xm)

  rows3 = pl.BlockSpec((AB_CHUNK, R2, R3), lambda i: (i, 0, 0))
  d2full = pl.BlockSpec((R2, R2), lambda i: (0, 0))
  t1spec = pl.BlockSpec((AB_CHUNK, R2), lambda i: (i, 0))
  d3full = pl.BlockSpec((R3, R3), lambda i: (0, 0))
  tbspec = pl.BlockSpec((R2, R3), lambda i: (0, 0))
  taspec = pl.BlockSpec((AB_CHUNK, R3), lambda i: (i, 0))
  ore, oim = pl.pallas_call(
      _fft23_body,
      grid=(S23_GRID,),
      in_specs=[d2full, d2full, t1spec, t1spec, d3full, d3full, d3full,
                tbspec, tbspec, taspec, taspec, rows3, rows3],
      out_specs=[rows3, rows3],
      out_shape=[jax.ShapeDtypeStruct((R1, R2, R3), jnp.float32)] * 2,
  )(jnp.asarray(D2C), jnp.asarray(D2S), jnp.asarray(T1C), jnp.asarray(T1S),
    jnp.asarray(D3C), jnp.asarray(D3S), jnp.asarray(D3C + D3S),
    jnp.asarray(TBC), jnp.asarray(TBS), jnp.asarray(TAC), jnp.asarray(TAS),
    yre, yim)
  return ore.reshape(R1 * R2, R3), oim.reshape(R1 * R2, R3)


NW = 32
TOT_ROWS = R1 * R2
ROWS_PER_W = TOT_ROWS // NW
R_CHUNK = 128
N_DMA = ROWS_PER_W // R_CHUNK
VALID_COLS = 64
SUB_ROWS = 8
N_SUB = R_CHUNK // SUB_ROWS
VEC = 16
VPR = VALID_COLS // VEC
N_VEC = SUB_ROWS * VPR


def _merge_pool(vm, vi, vre, vim, poolm, pooli, poolre, poolim, lane):
  nk, si, sre, sim = lax.sort((-vm, vi, vre, vim), num_keys=1)
  sm = -nk
  sel = lane < 8
  cm = jnp.where(sel, poolm, lax.rev(sm, (0,)))
  ci = jnp.where(sel, pooli, lax.rev(si, (0,)))
  cre = jnp.where(sel, poolre, lax.rev(sre, (0,)))
  cim = jnp.where(sel, poolim, lax.rev(sim, (0,)))
  nk2, pi2, pre2, pim2 = lax.sort((-cm, ci, cre, cim), num_keys=1)
  pm2 = -nk2
  new_t = jnp.min(jnp.where(lane < 5, pm2, jnp.float32(3e38)))
  return pm2, pi2, pre2, pim2, new_t


def _sc_topk_body(re_hbm, im_hbm, outm, outi, outre, outim,
                  re_buf, im_buf, stg_m, stg_i, stg_re, stg_im):
  wid = lax.axis_index("s") * 2 + lax.axis_index("c")
  row_base = wid * ROWS_PER_W
  lane = lax.iota(jnp.int32, 16)
  lane_k = lane << 16

  state0 = (
      jnp.full((VEC,), -1.0, jnp.float32),
      jnp.zeros((VEC,), jnp.int32),
      jnp.zeros((VEC,), jnp.float32),
      jnp.zeros((VEC,), jnp.float32),
      jnp.float32(-1.0),
  )

  def process_sub(sc, chunk_row0, state):
    r0 = sc * SUB_ROWS

    def maxbody(v, acc):
      r = r0 + (v >> 2)
      cc = (v & 3) * VEC
      rr = re_buf[r, pl.ds(cc, VEC)]
      ii = im_buf[r, pl.ds(cc, VEC)]
      return jnp.maximum(acc, rr * rr + ii * ii)

    m_acc = lax.fori_loop(0, N_VEC, maxbody, jnp.full((VEC,), -2.0, jnp.float32))
    sub_max = jnp.max(m_acc)

    def rescan(st):
      def body(v, st2):
        r = r0 + (v >> 2)
        cc = (v & 3) * VEC
        rr = re_buf[r, pl.ds(cc, VEC)]
        ii = im_buf[r, pl.ds(cc, VEC)]
        vm = rr * rr + ii * ii
        vmax = jnp.max(vm)

        def do_merge(st3):
          pm, pi, pre, pim, _t = st3
          grow = chunk_row0 + r
          a = grow >> 8
          b = grow & 255
          base_k = a + (b << 8) + (cc << 16)
          vi = lane_k + base_k
          return _merge_pool(vm, vi, rr, ii, pm, pi, pre, pim, lane)

        return lax.cond(vmax > st2[4], do_merge, lambda s: s, st2)

      return lax.fori_loop(0, N_VEC, body, st)

    return lax.cond(sub_max > state[4], rescan, lambda s: s, state)

  def dma_step(s, state):
    row0 = row_base + s * R_CHUNK
    pltpu.sync_copy(re_hbm.at[pl.ds(row0, R_CHUNK), pl.ds(0, VALID_COLS)],
                    re_buf)
    pltpu.sync_copy(im_hbm.at[pl.ds(row0, R_CHUNK), pl.ds(0, VALID_COLS)],
                    im_buf)

    @pl.when(jnp.logical_and(wid == 0, s == 0))
    def _zero_dc():
      rr = re_buf[0, pl.ds(0, VEC)]
      ii = im_buf[0, pl.ds(0, VEC)]
      re_buf[0, pl.ds(0, VEC)] = jnp.where(lane == 0, jnp.float32(0.0), rr)
      im_buf[0, pl.ds(0, VEC)] = jnp.where(lane == 0, jnp.float32(0.0), ii)

    def sub(c, st):
      return process_sub(c, row0, st)

    return lax.fori_loop(0, N_SUB, sub, state)

  state = lax.fori_loop(0, N_DMA, dma_step, state0)

  def nyq_merge(st):
    pltpu.sync_copy(re_hbm.at[0, pl.ds(VALID_COLS, VEC)], stg_re)
    pltpu.sync_copy(im_hbm.at[0, pl.ds(VALID_COLS, VEC)], stg_im)
    rr = stg_re[...]
    ii = stg_im[...]
    vm = jnp.where(lane == 0, rr * rr + ii * ii, jnp.float32(-1.0))
    vi = jnp.full((VEC,), NHALF, jnp.int32)
    pm, pi, pre, pim, _t = st
    return _merge_pool(vm, vi, rr, ii, pm, pi, pre, pim, lane)

  state = lax.cond(wid == 0, nyq_merge, lambda s: s, state)
  poolm, pooli, poolre, poolim, _t = state

  stg_m[...] = poolm
  stg_i[...] = pooli
  stg_re[...] = poolre
  stg_im[...] = poolim
  pltpu.sync_copy(stg_m, outm.at[pl.ds(wid * VEC, VEC)])
  pltpu.sync_copy(stg_i, outi.at[pl.ds(wid * VEC, VEC)])
  pltpu.sync_copy(stg_re, outre.at[pl.ds(wid * VEC, VEC)])
  pltpu.sync_copy(stg_im, outim.at[pl.ds(wid * VEC, VEC)])


@functools.cache
def _sc_topk():
  return pl.kernel(
      _sc_topk_body,
      out_type=[
          jax.ShapeDtypeStruct((NW * VEC,), jnp.float32),
          jax.ShapeDtypeStruct((NW * VEC,), jnp.int32),
          jax.ShapeDtypeStruct((NW * VEC,), jnp.float32),
          jax.ShapeDtypeStruct((NW * VEC,), jnp.float32),
      ],
      mesh=plsc.VectorSubcoreMesh(core_axis_name="c", subcore_axis_name="s"),
      compiler_params=pltpu.CompilerParams(needs_layout_passes=False,
                                           use_tc_tiling_on_sc=False),
      scratch_types=[
          pltpu.VMEM((R_CHUNK, VALID_COLS), jnp.float32),
          pltpu.VMEM((R_CHUNK, VALID_COLS), jnp.float32),
          pltpu.VMEM((VEC,), jnp.float32),
          pltpu.VMEM((VEC,), jnp.int32),
          pltpu.VMEM((VEC,), jnp.float32),
          pltpu.VMEM((VEC,), jnp.float32),
      ],
  )


ROWS = 8192
COLS = 1024
BLK_R = 256
GRID = ROWS // BLK_R


def _tc_synth_body(candm_ref, candi_ref, candre_ref, candim_ref, x_ref,
                   season_ref, trend_ref, uv_ref, jk_smem):
  i = pl.program_id(0)

  @pl.when(i == 0)
  def _init():
    m = candm_ref[...]
    ci = candi_ref[...]
    cre = candre_ref[...]
    cim = candim_ref[...]
    fi = (lax.broadcasted_iota(jnp.int32, m.shape, 0) * m.shape[1]
          + lax.broadcasted_iota(jnp.int32, m.shape, 1))
    avail = fi >= 0
    vals = []
    for _ in range(5):
      cur = jnp.where(avail, m, jnp.float32(-3.0))
      mk = jnp.max(cur)
      pick = jnp.min(jnp.where(cur == mk, fi, jnp.int32(1 << 30)))
      sel = fi == pick
      jk = jnp.sum(jnp.where(sel, ci, 0))
      rek = jnp.sum(jnp.where(sel, cre, jnp.float32(0.0)))
      imk = jnp.sum(jnp.where(sel, cim, jnp.float32(0.0)))
      avail = jnp.logical_and(avail, jnp.logical_not(sel))
      vals.append((mk, jk, rek, imk))
    thresh2 = vals[4][0]
    n2 = lax.broadcasted_iota(jnp.int32, (1, COLS), 1)
    zero_row = jnp.zeros((1, COLS), jnp.float32)
    for k in range(5):
      mk, jk, rek, imk = vals[k]
      alive = mk > thresh2
      is_nyq = jk == NHALF
      w = (jnp.where(is_nyq, jnp.float32(1.0), jnp.float32(2.0))
           * jnp.float32(1.0 / N))
      a = jnp.where(alive, w * rek, jnp.float32(0.0))
      b = jnp.where(jnp.logical_and(alive, jnp.logical_not(is_nyq)),
                    -w * imk, jnp.float32(0.0))
      m2 = (jk * n2) & PHASE_MASK
      th2 = m2.astype(jnp.float32) * jnp.float32(TWO_PI_OVER_N)
      c2 = jnp.cos(th2)
      s2 = jnp.sin(th2)
      uv_ref[pl.ds(k, 1), :] = a * c2 + b * s2
      uv_ref[pl.ds(8 + k, 1), :] = b * c2 - a * s2
      jk_smem[k] = jk
    for k in range(5, 8):
      uv_ref[pl.ds(k, 1), :] = zero_row
      uv_ref[pl.ds(8 + k, 1), :] = zero_row
      jk_smem[k] = 0

  n1k = (i * BLK_R + lax.broadcasted_iota(jnp.int32, (BLK_R, 1), 0)) * COLS
  jvec = jnp.concatenate(
      [lax.broadcast_in_dim(jk_smem[k], (1, 1), ()) for k in range(8)], axis=1)
  m1 = (jvec * n1k) & PHASE_MASK
  th1 = m1.astype(jnp.float32) * jnp.float32(TWO_PI_OVER_N)
  cs1 = jnp.concatenate([jnp.cos(th1), jnp.sin(th1)], axis=1)
  season = jnp.dot(cs1, uv_ref[...],
                   preferred_element_type=jnp.float32, precision=PREC)
  season_ref[...] = season
  trend_ref[...] = x_ref[...] - season


def _tc_synth(candm, candi, candre, candim, x2):
  cand_spec = pl.BlockSpec((4, 128), lambda i: (0, 0))
  return pl.pallas_call(
      _tc_synth_body,
      grid=(GRID,),
      in_specs=[cand_spec, cand_spec, cand_spec, cand_spec,
                pl.BlockSpec((BLK_R, COLS), lambda i: (i, 0))],
      out_specs=[pl.BlockSpec((BLK_R, COLS), lambda i: (i, 0)),
                 pl.BlockSpec((BLK_R, COLS), lambda i: (i, 0))],
      out_shape=[jax.ShapeDtypeStruct((ROWS, COLS), jnp.float32),
                 jax.ShapeDtypeStruct((ROWS, COLS), jnp.float32)],
      scratch_shapes=[pltpu.VMEM((16, COLS), jnp.float32),
                      pltpu.SMEM((8,), jnp.int32)],
  )(candm, candi, candre, candim, x2)


def kernel(x):
  ore, oim = _fft(x)
  candm, candi, candre, candim = _sc_topk()(ore, oim)
  x2 = x.reshape(ROWS, COLS)
  season, trend = _tc_synth(candm.reshape(4, 128), candi.reshape(4, 128),
                            candre.reshape(4, 128), candim.reshape(4, 128), x2)
  return season.reshape(-1), trend.reshape(-1)


# speedup vs baseline: 21.0226x; 1.1095x over previous
import functools

import numpy as np

import jax
import jax.numpy as jnp
from jax import lax
from jax.experimental import pallas as pl
from jax.experimental.pallas import tpu as pltpu
from jax.experimental.pallas import tpu_sc as plsc

N = 8388608
NHALF = N // 2
PHASE_MASK = N - 1
TWO_PI_OVER_N = 2.0 * np.pi / N

R1 = 256
R2 = 256
R3 = 128
N23 = R2 * R3

PREC = jax.lax.Precision.HIGHEST


def _dft_tables(r):
  k = np.arange(r)
  ph = 2.0 * np.pi * (np.outer(k, k) % r).astype(np.float64) / r
  return np.cos(ph).astype(np.float32), (-np.sin(ph)).astype(np.float32)

D1C, D1S = _dft_tables(R1)
D2C, D2S = _dft_tables(R2)
D3C, D3S = _dft_tables(R3)

_n2 = np.arange(R2)
_a = np.arange(R1)
_ph = 2.0 * np.pi * ((R3 * np.outer(_a, _n2)) % N).astype(np.float64) / N
T1C = np.cos(_ph).astype(np.float32)
T1S = (-np.sin(_ph)).astype(np.float32)

_c = np.arange(R3)
_ph = 2.0 * np.pi * (np.outer(_a, _c) % N).astype(np.float64) / N
TAC = np.cos(_ph).astype(np.float32)
TAS = (-np.sin(_ph)).astype(np.float32)
_b = np.arange(R2)
_ph = 2.0 * np.pi * (np.outer(_b, _c) % N23).astype(np.float64) / N23
TBC = np.cos(_ph).astype(np.float32)
TBS = (-np.sin(_ph)).astype(np.float32)


S1_COLS = 2048
S1_GRID = N23 // S1_COLS
AB_CHUNK = 8
S23_ROWS = AB_CHUNK * R2
S23_GRID = R1 // AB_CHUNK


N2_CHUNK = S1_COLS // R3


def _fft1_body(d1c_ref, d1s_ref, x_ref, yre_ref, yim_ref):
  xb = x_ref[...]
  yre = _dotd(d1c_ref[...], xb)
  yim = _dotd(d1s_ref[...], xb)
  for j in range(N2_CHUNK):
    yre_ref[:, j, :] = yre[:, j * R3:(j + 1) * R3]
    yim_ref[:, j, :] = yim[:, j * R3:(j + 1) * R3]


def _split(a):
  ah = a.astype(jnp.bfloat16)
  al = (a - ah.astype(jnp.float32)).astype(jnp.bfloat16)
  return ah, al


def _dot3s(a_split, b_split):
  ah, al = a_split
  bh, bl = b_split
  d = functools.partial(jnp.dot, preferred_element_type=jnp.float32)
  return d(ah, bh) + d(ah, bl) + d(al, bh)


def _dot3(a, b):
  return _dot3s(_split(a), _split(b))


def _dotd(a, b):
  return jnp.dot(a, b, preferred_element_type=jnp.float32)


def _fft23_body(d2c_ref, d2s_ref, t1c_ref, t1s_ref, d3c_ref, d3s_ref,
                d3cs_ref, tbc_ref, tbs_ref, tac_ref, tas_ref,
                yre_ref, yim_ref, ore_ref, oim_ref):
  d2c = d2c_ref[...]
  d2s = d2s_ref[...]
  d3c = d3c_ref[...]
  d3s = d3s_ref[...]
  d3cs = d3cs_ref[...]
  tbc = tbc_ref[...]
  tbs = tbs_ref[...]
  for j in range(AB_CHUNK):
    yre = yre_ref[j]
    yim = yim_ref[j]
    t1c = t1c_ref[pl.ds(j, 1), :]
    t1s = t1s_ref[pl.ds(j, 1), :]
    d2ca = d2c * t1c - d2s * t1s
    d2sa = d2c * t1s + d2s * t1c
    m1 = _dotd(d2ca, yre)
    m2 = _dotd(d2sa, yim)
    m3 = _dotd(d2ca + d2sa, yre + yim)
    zre = m1 - m2
    zim = m3 - m1 - m2
    tac = tac_ref[pl.ds(j, 1), :]
    tas = tas_ref[pl.ds(j, 1), :]
    t2c = tac * tbc - tas * tbs
    t2s = tac * tbs + tas * tbc
    ztre = zre * t2c - zim * t2s
    ztim = zre * t2s + zim * t2c
    k1 = _dotd(ztre, d3c)
    k2 = _dotd(ztim, d3s)
    k3 = _dotd(ztre + ztim, d3cs)
    ore_ref[j] = k1 - k2
    oim_ref[j] = k3 - k1 - k2


def _fft(x):
  xm = x.reshape(R1, N23)
  full = pl.BlockSpec((R1, R1), lambda i: (0, 0))
  y3 = pl.BlockSpec((R1, N2_CHUNK, R3), lambda i: (0, i, 0))
  yre, yim = pl.pallas_call(
      _fft1_body,
      grid=(S1_GRID,),
      in_specs=[full, full, pl.BlockSpec((R1, S1_COLS), lambda i: (0, i))],
      out_specs=[y3, y3],
      out_shape=[jax.ShapeDtypeStruct((R1, R2, R3), jnp.float32)] * 2,
  )(jnp.asarray(D1C), jnp.asarray(D1S), xm)

  rows3 = pl.BlockSpec((AB_CHUNK, R2, R3), lambda i: (i, 0, 0))
  d2full = pl.BlockSpec((R2, R2), lambda i: (0, 0))
  t1spec = pl.BlockSpec((AB_CHUNK, R2), lambda i: (i, 0))
  d3full = pl.BlockSpec((R3, R3), lambda i: (0, 0))
  tbspec = pl.BlockSpec((R2, R3), lambda i: (0, 0))
  taspec = pl.BlockSpec((AB_CHUNK, R3), lambda i: (i, 0))
  ore, oim = pl.pallas_call(
      _fft23_body,
      grid=(S23_GRID,),
      in_specs=[d2full, d2full, t1spec, t1spec, d3full, d3full, d3full,
                tbspec, tbspec, taspec, taspec, rows3, rows3],
      out_specs=[rows3, rows3],
      out_shape=[jax.ShapeDtypeStruct((R1, R2, R3), jnp.float32)] * 2,
  )(jnp.asarray(D2C), jnp.asarray(D2S), jnp.asarray(T1C), jnp.asarray(T1S),
    jnp.asarray(D3C), jnp.asarray(D3S), jnp.asarray(D3C + D3S),
    jnp.asarray(TBC), jnp.asarray(TBS), jnp.asarray(TAC), jnp.asarray(TAS),
    yre, yim)
  return ore.reshape(R1 * R2, R3), oim.reshape(R1 * R2, R3)


NW = 32
TOT_ROWS = R1 * R2
ROWS_PER_W = TOT_ROWS // NW
R_CHUNK = 128
N_DMA = ROWS_PER_W // R_CHUNK
VALID_COLS = 64
SUB_ROWS = 8
N_SUB = R_CHUNK // SUB_ROWS
VEC = 16
VPR = VALID_COLS // VEC
N_VEC = SUB_ROWS * VPR


def _merge_pool(vm, vi, vre, vim, poolm, pooli, poolre, poolim, lane):
  nk, si, sre, sim = lax.sort((-vm, vi, vre, vim), num_keys=1)
  sm = -nk
  sel = lane < 8
  cm = jnp.where(sel, poolm, lax.rev(sm, (0,)))
  ci = jnp.where(sel, pooli, lax.rev(si, (0,)))
  cre = jnp.where(sel, poolre, lax.rev(sre, (0,)))
  cim = jnp.where(sel, poolim, lax.rev(sim, (0,)))
  nk2, pi2, pre2, pim2 = lax.sort((-cm, ci, cre, cim), num_keys=1)
  pm2 = -nk2
  new_t = jnp.min(jnp.where(lane < 5, pm2, jnp.float32(3e38)))
  return pm2, pi2, pre2, pim2, new_t


def _sc_topk_body(re_hbm, im_hbm, outm, outi, outre, outim,
                  re_buf, im_buf, stg_m, stg_i, stg_re, stg_im):
  wid = lax.axis_index("s") * 2 + lax.axis_index("c")
  row_base = wid * ROWS_PER_W
  lane = lax.iota(jnp.int32, 16)
  lane_k = lane << 16

  state0 = (
      jnp.full((VEC,), -1.0, jnp.float32),
      jnp.zeros((VEC,), jnp.int32),
      jnp.zeros((VEC,), jnp.float32),
      jnp.zeros((VEC,), jnp.float32),
      jnp.float32(-1.0),
  )

  def process_sub(sc, chunk_row0, state):
    r0 = sc * SUB_ROWS

    def maxbody(v, acc):
      r = r0 + (v >> 2)
      cc = (v & 3) * VEC
      rr = re_buf[r, pl.ds(cc, VEC)]
      ii = im_buf[r, pl.ds(cc, VEC)]
      return jnp.maximum(acc, rr * rr + ii * ii)

    m_acc = lax.fori_loop(0, N_VEC, maxbody, jnp.full((VEC,), -2.0, jnp.float32))
    sub_max = jnp.max(m_acc)

    def rescan(st):
      def body(v, st2):
        r = r0 + (v >> 2)
        cc = (v & 3) * VEC
        rr = re_buf[r, pl.ds(cc, VEC)]
        ii = im_buf[r, pl.ds(cc, VEC)]
        vm = rr * rr + ii * ii
        vmax = jnp.max(vm)

        def do_merge(st3):
          pm, pi, pre, pim, _t = st3
          grow = chunk_row0 + r
          a = grow >> 8
          b = grow & 255
          base_k = a + (b << 8) + (cc << 16)
          vi = lane_k + base_k
          return _merge_pool(vm, vi, rr, ii, pm, pi, pre, pim, lane)

        return lax.cond(vmax > st2[4], do_merge, lambda s: s, st2)

      return lax.fori_loop(0, N_VEC, body, st)

    return lax.cond(sub_max > state[4], rescan, lambda s: s, state)

  def dma_step(s, state):
    row0 = row_base + s * R_CHUNK
    pltpu.sync_copy(re_hbm.at[pl.ds(row0, R_CHUNK), pl.ds(0, VALID_COLS)],
                    re_buf)
    pltpu.sync_copy(im_hbm.at[pl.ds(row0, R_CHUNK), pl.ds(0, VALID_COLS)],
                    im_buf)

    @pl.when(jnp.logical_and(wid == 0, s == 0))
    def _zero_dc():
      rr = re_buf[0, pl.ds(0, VEC)]
      ii = im_buf[0, pl.ds(0, VEC)]
      re_buf[0, pl.ds(0, VEC)] = jnp.where(lane == 0, jnp.float32(0.0), rr)
      im_buf[0, pl.ds(0, VEC)] = jnp.where(lane == 0, jnp.float32(0.0), ii)

    def sub(c, st):
      return process_sub(c, row0, st)

    return lax.fori_loop(0, N_SUB, sub, state)

  state = lax.fori_loop(0, N_DMA, dma_step, state0)

  def nyq_merge(st):
    pltpu.sync_copy(re_hbm.at[0, pl.ds(VALID_COLS, VEC)], stg_re)
    pltpu.sync_copy(im_hbm.at[0, pl.ds(VALID_COLS, VEC)], stg_im)
    rr = stg_re[...]
    ii = stg_im[...]
    vm = jnp.where(lane == 0, rr * rr + ii * ii, jnp.float32(-1.0))
    vi = jnp.full((VEC,), NHALF, jnp.int32)
    pm, pi, pre, pim, _t = st
    return _merge_pool(vm, vi, rr, ii, pm, pi, pre, pim, lane)

  state = lax.cond(wid == 0, nyq_merge, lambda s: s, state)
  poolm, pooli, poolre, poolim, _t = state

  stg_m[...] = poolm
  stg_i[...] = pooli
  stg_re[...] = poolre
  stg_im[...] = poolim
  pltpu.sync_copy(stg_m, outm.at[pl.ds(wid * VEC, VEC)])
  pltpu.sync_copy(stg_i, outi.at[pl.ds(wid * VEC, VEC)])
  pltpu.sync_copy(stg_re, outre.at[pl.ds(wid * VEC, VEC)])
  pltpu.sync_copy(stg_im, outim.at[pl.ds(wid * VEC, VEC)])


@functools.cache
def _sc_topk():
  return pl.kernel(
      _sc_topk_body,
      out_type=[
          jax.ShapeDtypeStruct((NW * VEC,), jnp.float32),
          jax.ShapeDtypeStruct((NW * VEC,), jnp.int32),
          jax.ShapeDtypeStruct((NW * VEC,), jnp.float32),
          jax.ShapeDtypeStruct((NW * VEC,), jnp.float32),
      ],
      mesh=plsc.VectorSubcoreMesh(core_axis_name="c", subcore_axis_name="s"),
      compiler_params=pltpu.CompilerParams(needs_layout_passes=False,
                                           use_tc_tiling_on_sc=False),
      scratch_types=[
          pltpu.VMEM((R_CHUNK, VALID_COLS), jnp.float32),
          pltpu.VMEM((R_CHUNK, VALID_COLS), jnp.float32),
          pltpu.VMEM((VEC,), jnp.float32),
          pltpu.VMEM((VEC,), jnp.int32),
          pltpu.VMEM((VEC,), jnp.float32),
          pltpu.VMEM((VEC,), jnp.float32),
      ],
  )


ROWS = 8192
COLS = 1024
BLK_R = 256
GRID = ROWS // BLK_R
NREF = 16


def _refine_body(candm_ref, candi_ref, x_ref, outre_ref, outim_ref, outj_ref,
                 e2ct, e2st, accre, accim, jk_smem):
  i = pl.program_id(0)

  @pl.when(i == 0)
  def _init():
    m = candm_ref[...]
    ci = candi_ref[...]
    fi = (lax.broadcasted_iota(jnp.int32, m.shape, 0) * m.shape[1]
          + lax.broadcasted_iota(jnp.int32, m.shape, 1))
    avail = fi >= 0
    for k in range(NREF):
      cur = jnp.where(avail, m, jnp.float32(-3.0))
      mk = jnp.max(cur)
      pick = jnp.min(jnp.where(cur == mk, fi, jnp.int32(1 << 30)))
      sel = fi == pick
      jk_smem[k] = jnp.sum(jnp.where(sel, ci, 0))
      avail = jnp.logical_and(avail, jnp.logical_not(sel))
    jv = jnp.concatenate(
        [lax.broadcast_in_dim(jk_smem[k], (1, 1), ()) for k in range(NREF)],
        axis=1)
    n2c = lax.broadcasted_iota(jnp.int32, (COLS, 1), 0)
    m2 = (jv * n2c) & PHASE_MASK
    th2 = m2.astype(jnp.float32) * jnp.float32(TWO_PI_OVER_N)
    e2ct[...] = jnp.cos(th2)
    e2st[...] = jnp.sin(th2)
    accre[...] = jnp.zeros((1, NREF), jnp.float32)
    accim[...] = jnp.zeros((1, NREF), jnp.float32)

  jv = jnp.concatenate(
      [lax.broadcast_in_dim(jk_smem[k], (1, 1), ()) for k in range(NREF)],
      axis=1)
  xb = x_ref[...]
  xs = _split(xb)
  p = _dot3s(xs, _split(e2ct[...]))
  q = _dot3s(xs, _split(e2st[...]))
  n1k = (i * BLK_R + lax.broadcasted_iota(jnp.int32, (BLK_R, 1), 0)) * COLS
  m1 = (jv * n1k) & PHASE_MASK
  th1 = m1.astype(jnp.float32) * jnp.float32(TWO_PI_OVER_N)
  c1 = jnp.cos(th1)
  s1 = jnp.sin(th1)
  accre[...] = accre[...] + jnp.sum(c1 * p - s1 * q, axis=0, keepdims=True)
  accim[...] = accim[...] - jnp.sum(s1 * p + c1 * q, axis=0, keepdims=True)

  @pl.when(i == GRID - 1)
  def _fin():
    outre_ref[...] = accre[...]
    outim_ref[...] = accim[...]
    outj_ref[...] = jv


def _refine(candm, candi, x2):
  cand_spec = pl.BlockSpec((4, 128), lambda i: (0, 0))
  out_spec = pl.BlockSpec((1, NREF), lambda i: (0, 0))
  return pl.pallas_call(
      _refine_body,
      grid=(GRID,),
      in_specs=[cand_spec, cand_spec,
                pl.BlockSpec((BLK_R, COLS), lambda i: (i, 0))],
      out_specs=[out_spec, out_spec, out_spec],
      out_shape=[jax.ShapeDtypeStruct((1, NREF), jnp.float32),
                 jax.ShapeDtypeStruct((1, NREF), jnp.float32),
                 jax.ShapeDtypeStruct((1, NREF), jnp.int32)],
      scratch_shapes=[pltpu.VMEM((COLS, NREF), jnp.float32),
                      pltpu.VMEM((COLS, NREF), jnp.float32),
                      pltpu.VMEM((1, NREF), jnp.float32),
                      pltpu.VMEM((1, NREF), jnp.float32),
                      pltpu.SMEM((NREF,), jnp.int32)],
  )(candm, candi, x2)


def _tc_synth_body(candre_ref, candim_ref, candj_ref, x_ref,
                   season_ref, trend_ref, uv_ref, jk_smem):
  i = pl.program_id(0)

  @pl.when(i == 0)
  def _init():
    cre = candre_ref[...]
    cim = candim_ref[...]
    ci = candj_ref[...]
    m = cre * cre + cim * cim
    fi = (lax.broadcasted_iota(jnp.int32, m.shape, 0) * m.shape[1]
          + lax.broadcasted_iota(jnp.int32, m.shape, 1))
    avail = fi >= 0
    vals = []
    for _ in range(5):
      cur = jnp.where(avail, m, jnp.float32(-3.0))
      mk = jnp.max(cur)
      pick = jnp.min(jnp.where(cur == mk, fi, jnp.int32(1 << 30)))
      sel = fi == pick
      jk = jnp.sum(jnp.where(sel, ci, 0))
      rek = jnp.sum(jnp.where(sel, cre, jnp.float32(0.0)))
      imk = jnp.sum(jnp.where(sel, cim, jnp.float32(0.0)))
      avail = jnp.logical_and(avail, jnp.logical_not(sel))
      vals.append((mk, jk, rek, imk))
    thresh2 = vals[4][0]
    n2 = lax.broadcasted_iota(jnp.int32, (1, COLS), 1)
    zero_row = jnp.zeros((1, COLS), jnp.float32)
    for k in range(5):
      mk, jk, rek, imk = vals[k]
      alive = mk > thresh2
      is_nyq = jk == NHALF
      w = (jnp.where(is_nyq, jnp.float32(1.0), jnp.float32(2.0))
           * jnp.float32(1.0 / N))
      a = jnp.where(alive, w * rek, jnp.float32(0.0))
      b = jnp.where(jnp.logical_and(alive, jnp.logical_not(is_nyq)),
                    -w * imk, jnp.float32(0.0))
      m2 = (jk * n2) & PHASE_MASK
      th2 = m2.astype(jnp.float32) * jnp.float32(TWO_PI_OVER_N)
      c2 = jnp.cos(th2)
      s2 = jnp.sin(th2)
      uv_ref[pl.ds(k, 1), :] = a * c2 + b * s2
      uv_ref[pl.ds(8 + k, 1), :] = b * c2 - a * s2
      jk_smem[k] = jk
    for k in range(5, 8):
      uv_ref[pl.ds(k, 1), :] = zero_row
      uv_ref[pl.ds(8 + k, 1), :] = zero_row
      jk_smem[k] = 0

  n1k = (i * BLK_R + lax.broadcasted_iota(jnp.int32, (BLK_R, 1), 0)) * COLS
  jvec = jnp.concatenate(
      [lax.broadcast_in_dim(jk_smem[k], (1, 1), ()) for k in range(8)], axis=1)
  m1 = (jvec * n1k) & PHASE_MASK
  th1 = m1.astype(jnp.float32) * jnp.float32(TWO_PI_OVER_N)
  cs1 = jnp.concatenate([jnp.cos(th1), jnp.sin(th1)], axis=1)
  season = jnp.dot(cs1, uv_ref[...],
                   preferred_element_type=jnp.float32, precision=PREC)
  season_ref[...] = season
  trend_ref[...] = x_ref[...] - season


def _tc_synth(candre, candim, candj, x2):
  cand_spec = pl.BlockSpec((1, NREF), lambda i: (0, 0))
  return pl.pallas_call(
      _tc_synth_body,
      grid=(GRID,),
      in_specs=[cand_spec, cand_spec, cand_spec,
                pl.BlockSpec((BLK_R, COLS), lambda i: (i, 0))],
      out_specs=[pl.BlockSpec((BLK_R, COLS), lambda i: (i, 0)),
                 pl.BlockSpec((BLK_R, COLS), lambda i: (i, 0))],
      out_shape=[jax.ShapeDtypeStruct((ROWS, COLS), jnp.float32),
                 jax.ShapeDtypeStruct((ROWS, COLS), jnp.float32)],
      scratch_shapes=[pltpu.VMEM((16, COLS), jnp.float32),
                      pltpu.SMEM((8,), jnp.int32)],
  )(candre, candim, candj, x2)


def kernel(x):
  ore, oim = _fft(x)
  candm, candi, _candre, _candim = _sc_topk()(ore, oim)
  x2 = x.reshape(ROWS, COLS)
  refre, refim, refj = _refine(candm.reshape(4, 128), candi.reshape(4, 128), x2)
  season, trend = _tc_synth(refre, refim, refj, x2)
  return season.reshape(-1), trend.reshape(-1)
